# Initial kernel scaffold; baseline (speedup 1.0000x reference)
import jax
import jax.numpy as jnp
from jax.experimental import pallas as pl


def kernel(nodes, edges, globals_, receivers, senders, W, b):
    raise NotImplementedError("write your pallas kernel here")


# speedup vs baseline: 3.1061x; 3.1061x over previous
import functools

import jax
import jax.numpy as jnp
from jax import lax
from jax.experimental import pallas as pl
from jax.experimental.pallas import tpu as pltpu
from jax.experimental.pallas import tpu_sc as plsc

N_NODES = 10000
N_EDGES = 160000
D_NODE = 256
D_EDGE = 16
D_GLOBAL = 64

_NC = 2
_NS = 16
_NW = _NC * _NS
_EPW = N_EDGES // _NW
_CHUNK = 1000
_NCHUNK = _EPW // _CHUNK


def _project_body(nodes_ref, wr_ref, ws_ref, pr_ref, ps_ref):
    n = nodes_ref[...]
    dn = (((1,), (1,)), ((), ()))
    pr_ref[...] = lax.dot_general(n, wr_ref[...], dn,
                                  preferred_element_type=jnp.float32)
    ps_ref[...] = lax.dot_general(n, ws_ref[...], dn,
                                  preferred_element_type=jnp.float32)


def _project(nodes, wr, ws):
    blk = 2000
    grid = (N_NODES // blk,)
    return pl.pallas_call(
        _project_body,
        grid=grid,
        in_specs=[
            pl.BlockSpec((blk, D_NODE), lambda i: (i, 0)),
            pl.BlockSpec((D_EDGE, D_NODE), lambda i: (0, 0)),
            pl.BlockSpec((D_EDGE, D_NODE), lambda i: (0, 0)),
        ],
        out_specs=[
            pl.BlockSpec((blk, D_EDGE), lambda i: (i, 0)),
            pl.BlockSpec((blk, D_EDGE), lambda i: (i, 0)),
        ],
        out_shape=[
            jax.ShapeDtypeStruct((N_NODES, D_EDGE), jnp.float32),
            jax.ShapeDtypeStruct((N_NODES, D_EDGE), jnp.float32),
        ],
    )(nodes, wr, ws)


def _base_body(edges_ref, we_ref, g_ref, wg_ref, b_ref, out_ref):
    dn = (((1,), (1,)), ((), ()))
    const = lax.dot_general(g_ref[...], wg_ref[...], dn,
                            preferred_element_type=jnp.float32) + b_ref[...]
    out_ref[...] = lax.dot_general(edges_ref[...], we_ref[...], dn,
                                   preferred_element_type=jnp.float32) + const


def _edge_base(edges, we, g, wg, b2):
    blk = 16000
    grid = (N_EDGES // blk,)
    return pl.pallas_call(
        _base_body,
        grid=grid,
        in_specs=[
            pl.BlockSpec((blk, D_EDGE), lambda i: (i, 0)),
            pl.BlockSpec((D_EDGE, D_EDGE), lambda i: (0, 0)),
            pl.BlockSpec((1, D_GLOBAL), lambda i: (0, 0)),
            pl.BlockSpec((D_EDGE, D_GLOBAL), lambda i: (0, 0)),
            pl.BlockSpec((1, D_EDGE), lambda i: (0, 0)),
        ],
        out_specs=pl.BlockSpec((blk, D_EDGE), lambda i: (i, 0)),
        out_shape=jax.ShapeDtypeStruct((N_EDGES, D_EDGE), jnp.float32),
    )(edges, we, g, wg, b2)


def _make_combine():
    mesh = plsc.VectorSubcoreMesh(core_axis_name="c", subcore_axis_name="s")

    @functools.partial(
        pl.kernel,
        mesh=mesh,
        out_type=jax.ShapeDtypeStruct((N_EDGES, D_EDGE), jnp.float32),
        compiler_params=pltpu.CompilerParams(use_tc_tiling_on_sc=False),
        scratch_types=[
            pltpu.VMEM((_CHUNK,), jnp.int32),
            pltpu.VMEM((_CHUNK,), jnp.int32),
            pltpu.VMEM((_CHUNK, D_EDGE), jnp.float32),
            pltpu.VMEM((_CHUNK, D_EDGE), jnp.float32),
            pltpu.VMEM((_CHUNK, D_EDGE), jnp.float32),
            pltpu.SemaphoreType.DMA,
            pltpu.SemaphoreType.DMA,
        ],
    )
    def combine(pr_hbm, ps_hbm, base_hbm, recv_hbm, send_hbm, out_hbm,
                idxr, idxs, rowsr, rowss, acc, semr, sems):
        wid = lax.axis_index("s") * _NC + lax.axis_index("c")
        ebase = wid * _EPW

        def chunk(ci, carry):
            off = ebase + ci * _CHUNK
            pltpu.sync_copy(recv_hbm.at[pl.ds(off, _CHUNK)], idxr)
            pltpu.sync_copy(send_hbm.at[pl.ds(off, _CHUNK)], idxs)
            cr = pltpu.async_copy(pr_hbm.at[idxr], rowsr, semr)
            cs = pltpu.async_copy(ps_hbm.at[idxs], rowss, sems)
            pltpu.sync_copy(base_hbm.at[pl.ds(off, _CHUNK)], acc)
            cr.wait()
            cs.wait()

            def row(i, c2):
                acc[i, :] = acc[i, :] + rowsr[i, :] + rowss[i, :]
                return c2

            lax.fori_loop(0, _CHUNK, row, 0)
            pltpu.sync_copy(acc, out_hbm.at[pl.ds(off, _CHUNK)])
            return carry

        lax.fori_loop(0, _NCHUNK, chunk, 0)

    return combine


_combine = _make_combine()


def kernel(nodes, edges, globals_, receivers, senders, W, b):
    we = W[:, :D_EDGE]
    wr = W[:, D_EDGE:D_EDGE + D_NODE]
    ws = W[:, D_EDGE + D_NODE:D_EDGE + 2 * D_NODE]
    wg = W[:, D_EDGE + 2 * D_NODE:]
    pr, ps = _project(nodes, wr, ws)
    base = _edge_base(edges, we, globals_, wg, b.reshape(1, D_EDGE))
    return _combine(pr, ps, base, receivers, senders)
